# baseline (device time: 103317 ns/iter reference)
import jax
import jax.numpy as jnp
from jax import lax
from jax.experimental import pallas as pl
from jax.experimental.pallas import tpu as pltpu

N_DEV = 4
SCALE = 0.08838834764831843
GQA_REP = 4


def kernel(x, Wq, Wo, K_ext, V_ext):
    B, Sq, D = x.shape
    Dq = Wq.shape[1]
    Dh = K_ext.shape[-1]
    n_local_heads = Dq // Dh
    n_local_kv = n_local_heads // GQA_REP

    my_i = lax.axis_index("i")
    xs = x[0]
    Kl = lax.dynamic_slice_in_dim(K_ext[0], n_local_kv * my_i, n_local_kv, axis=1)
    Vl = lax.dynamic_slice_in_dim(V_ext[0], n_local_kv * my_i, n_local_kv, axis=1)
    Kl = Kl.transpose(1, 0, 2)
    Vl = Vl.transpose(1, 0, 2)

    def body(x_ref, wq_ref, wo_ref, k_ref, v_ref, out_ref, comm_ref,
             send_sems, recv_sems):
        my_pos = lax.axis_index("i")
        left = lax.rem(my_pos + N_DEV - 1, N_DEV)
        right = lax.rem(my_pos + 1, N_DEV)

        barrier_sem = pltpu.get_barrier_semaphore()
        for nbr in (left, right):
            pl.semaphore_signal(barrier_sem, inc=1, device_id=(nbr,),
                                device_id_type=pl.DeviceIdType.MESH)
        pl.semaphore_wait(barrier_sem, 2)

        q = jnp.dot(x_ref[:, :], wq_ref[:, :],
                    preferred_element_type=jnp.float32)
        outs = []
        for h in range(n_local_heads):
            g = h // GQA_REP
            q_h = q[:, h * Dh:(h + 1) * Dh]
            s = lax.dot_general(q_h, k_ref[g], (((1,), (1,)), ((), ())),
                                preferred_element_type=jnp.float32) * SCALE
            m = jnp.max(s, axis=1, keepdims=True)
            p = jnp.exp(s - m)
            l = jnp.sum(p, axis=1, keepdims=True)
            o = jnp.dot(p, v_ref[g], preferred_element_type=jnp.float32) / l
            outs.append(o)
        attn = jnp.concatenate(outs, axis=1)
        partial = jnp.dot(attn, wo_ref[:, :],
                          preferred_element_type=jnp.float32)

        out_ref[:, :] = partial
        comm_ref[0] = partial

        for h in range(N_DEV - 1):
            send_slot = h % 2
            recv_slot = (h + 1) % 2
            rdma = pltpu.make_async_remote_copy(
                src_ref=comm_ref.at[send_slot],
                dst_ref=comm_ref.at[recv_slot],
                send_sem=send_sems.at[send_slot],
                recv_sem=recv_sems.at[recv_slot],
                device_id=(right,),
                device_id_type=pl.DeviceIdType.MESH,
            )
            rdma.start()
            rdma.wait()
            out_ref[:, :] += comm_ref[recv_slot]

    out = pl.pallas_call(
        body,
        out_shape=jax.ShapeDtypeStruct((Sq, D), jnp.float32),
        in_specs=[pl.BlockSpec(memory_space=pltpu.VMEM)] * 5,
        out_specs=pl.BlockSpec(memory_space=pltpu.VMEM),
        scratch_shapes=[
            pltpu.VMEM((2, Sq, D), jnp.float32),
            pltpu.SemaphoreType.DMA((2,)),
            pltpu.SemaphoreType.DMA((2,)),
        ],
        compiler_params=pltpu.CompilerParams(collective_id=0),
    )(xs, Wq, Wo, Kl, Vl)
    return out.reshape(B, Sq, D)


# device time: 57117 ns/iter; 1.8089x vs baseline; 1.8089x over previous
import jax
import jax.numpy as jnp
from jax import lax
from jax.experimental import pallas as pl
from jax.experimental.pallas import tpu as pltpu

N_DEV = 4
SCALE = 0.08838834764831843
GQA_REP = 4


def kernel(x, Wq, Wo, K_ext, V_ext):
    B, Sq, D = x.shape
    Dq = Wq.shape[1]
    Dh = K_ext.shape[-1]
    n_local_heads = Dq // Dh
    n_local_kv = n_local_heads // GQA_REP

    my_i = lax.axis_index("i")
    xs = x[0]
    Kl = lax.dynamic_slice_in_dim(K_ext[0], n_local_kv * my_i, n_local_kv, axis=1)
    Vl = lax.dynamic_slice_in_dim(V_ext[0], n_local_kv * my_i, n_local_kv, axis=1)
    Kl = Kl.transpose(1, 0, 2)
    Vl = Vl.transpose(1, 0, 2)

    def body(x_ref, wq_ref, wo_ref, k_ref, v_ref, out_ref,
             stage_p, stage_m, send_p, recv_p, send_m, recv_m):
        my_pos = lax.axis_index("i")
        left = lax.rem(my_pos + N_DEV - 1, N_DEV)
        right = lax.rem(my_pos + 1, N_DEV)

        barrier_sem = pltpu.get_barrier_semaphore()
        for nbr in (left, right):
            pl.semaphore_signal(barrier_sem, inc=1, device_id=(nbr,),
                                device_id_type=pl.DeviceIdType.MESH)
        pl.semaphore_wait(barrier_sem, 2)

        q = jnp.dot(x_ref[:, :], wq_ref[:, :],
                    preferred_element_type=jnp.float32)
        outs = []
        for h in range(n_local_heads):
            g = h // GQA_REP
            q_h = q[:, h * Dh:(h + 1) * Dh]
            s = lax.dot_general(q_h, k_ref[g], (((1,), (1,)), ((), ())),
                                preferred_element_type=jnp.float32) * SCALE
            m = jnp.max(s, axis=1, keepdims=True)
            p = jnp.exp(s - m)
            l = jnp.sum(p, axis=1, keepdims=True)
            o = jnp.dot(p, v_ref[g], preferred_element_type=jnp.float32) / l
            outs.append(o)
        attn = jnp.concatenate(outs, axis=1)
        partial = jnp.dot(attn, wo_ref[:, :],
                          preferred_element_type=jnp.float32)

        out_ref[:, :] = partial

        CH = Sq // (2 * N_DEV)
        HB = Sq // 2
        p = my_pos

        def mod4(v):
            return lax.rem(v + 2 * N_DEV, N_DEV)

        for s in range(N_DEV - 1):
            cs_p, cr_p = mod4(p - s), mod4(p - s - 1)
            cs_m, cr_m = mod4(p + s), mod4(p + s + 1)
            r_p = pltpu.make_async_remote_copy(
                src_ref=out_ref.at[pl.ds(CH * cs_p, CH), :],
                dst_ref=stage_p.at[s],
                send_sem=send_p.at[s], recv_sem=recv_p.at[s],
                device_id=(right,), device_id_type=pl.DeviceIdType.MESH,
            )
            r_m = pltpu.make_async_remote_copy(
                src_ref=out_ref.at[pl.ds(HB + CH * cs_m, CH), :],
                dst_ref=stage_m.at[s],
                send_sem=send_m.at[s], recv_sem=recv_m.at[s],
                device_id=(left,), device_id_type=pl.DeviceIdType.MESH,
            )
            r_p.start()
            r_m.start()
            r_p.wait()
            r_m.wait()
            out_ref[pl.ds(CH * cr_p, CH), :] += stage_p[s]
            out_ref[pl.ds(HB + CH * cr_m, CH), :] += stage_m[s]

        for s in range(N_DEV - 1):
            k = N_DEV - 1 + s
            cs_p, cr_p = mod4(p + 1 - s), mod4(p - s)
            cs_m, cr_m = mod4(p - 1 + s), mod4(p + s)
            r_p = pltpu.make_async_remote_copy(
                src_ref=out_ref.at[pl.ds(CH * cs_p, CH), :],
                dst_ref=out_ref.at[pl.ds(CH * cs_p, CH), :],
                send_sem=send_p.at[k], recv_sem=recv_p.at[k],
                device_id=(right,), device_id_type=pl.DeviceIdType.MESH,
            )
            r_m = pltpu.make_async_remote_copy(
                src_ref=out_ref.at[pl.ds(HB + CH * cs_m, CH), :],
                dst_ref=out_ref.at[pl.ds(HB + CH * cs_m, CH), :],
                send_sem=send_m.at[k], recv_sem=recv_m.at[k],
                device_id=(left,), device_id_type=pl.DeviceIdType.MESH,
            )
            r_p.start()
            r_m.start()
            r_p.wait()
            r_m.wait()

    out = pl.pallas_call(
        body,
        out_shape=jax.ShapeDtypeStruct((Sq, D), jnp.float32),
        in_specs=[pl.BlockSpec(memory_space=pltpu.VMEM)] * 5,
        out_specs=pl.BlockSpec(memory_space=pltpu.VMEM),
        scratch_shapes=[
            pltpu.VMEM((N_DEV - 1, Sq // (2 * N_DEV), D), jnp.float32),
            pltpu.VMEM((N_DEV - 1, Sq // (2 * N_DEV), D), jnp.float32),
            pltpu.SemaphoreType.DMA((2 * (N_DEV - 1),)),
            pltpu.SemaphoreType.DMA((2 * (N_DEV - 1),)),
            pltpu.SemaphoreType.DMA((2 * (N_DEV - 1),)),
            pltpu.SemaphoreType.DMA((2 * (N_DEV - 1),)),
        ],
        compiler_params=pltpu.CompilerParams(collective_id=0),
    )(xs, Wq, Wo, Kl, Vl)
    return out.reshape(B, Sq, D)
